# baseline (speedup 1.0000x reference)
import functools

import jax
import jax.numpy as jnp
from jax import lax
from jax.experimental import pallas as pl
from jax.experimental.pallas import tpu as pltpu

T, H, I, E = 2048, 2048, 1536, 8
NE = E + 1
EPAD = 128
TB = 256
TI = 512
NT = T // TB
NI = I // TI


def _router_body(x_ref, gw_ref, comb_ref):
    xb = x_ref[...]
    gw = gw_ref[...]
    logits = lax.dot_general(xb, gw, (((1,), (1,)), ((), ())),
                             preferred_element_type=jnp.float32)
    lane = lax.broadcasted_iota(jnp.int32, (TB, EPAD), 1)
    neg = jnp.float32(-1e30)
    l = jnp.where(lane < E, logits, neg)
    m0 = jnp.max(l, axis=1, keepdims=True)
    i0 = jnp.min(jnp.where(l == m0, lane, EPAD), axis=1, keepdims=True)
    l2 = jnp.where(lane == i0, neg, l)
    m1 = jnp.max(l2, axis=1, keepdims=True)
    i1 = jnp.min(jnp.where(l2 == m1, lane, EPAD), axis=1, keepdims=True)
    w0 = jax.nn.sigmoid(m0 - m1)
    w1 = 1.0 - w0
    comb = (jnp.where(lane == i0, w0, 0.0)
            + jnp.where(lane == i1, w1, 0.0)
            + jnp.where(lane == E, 1.0, 0.0))
    comb_ref[...] = comb


def _moe_body(x_ref, wg_ref, wu_ref, wd_ref, comb_ref, out_ref):
    e = pl.program_id(1)
    i = pl.program_id(2)
    xb = x_ref[...]
    wg = wg_ref[0]
    wu = wu_ref[0]
    g = lax.dot_general(xb, wg, (((1,), (1,)), ((), ())),
                        preferred_element_type=jnp.float32)
    u = lax.dot_general(xb, wu, (((1,), (1,)), ((), ())),
                        preferred_element_type=jnp.float32)
    h = (g * jax.nn.sigmoid(g) * u).astype(jnp.bfloat16)
    wd = wd_ref[0]
    partial = lax.dot_general(h, wd, (((1,), (1,)), ((), ())),
                              preferred_element_type=jnp.float32)
    lane = lax.broadcasted_iota(jnp.int32, (TB, EPAD), 1)
    w = jnp.sum(jnp.where(lane == e, comb_ref[...], 0.0), axis=1,
                keepdims=True)
    contrib = partial * w

    @pl.when(jnp.logical_and(e == 0, i == 0))
    def _init():
        out_ref[...] = contrib

    @pl.when(jnp.logical_not(jnp.logical_and(e == 0, i == 0)))
    def _acc():
        out_ref[...] += contrib


@functools.partial(jax.jit, static_argnames=())
def kernel(x, gate_w, Wg, Wu, Wd, sg, su, sd):
    gw_pad = jnp.zeros((EPAD, H), jnp.float32).at[:E].set(gate_w)
    combine = pl.pallas_call(
        _router_body,
        grid=(NT,),
        in_specs=[
            pl.BlockSpec((TB, H), lambda t: (t, 0)),
            pl.BlockSpec((EPAD, H), lambda t: (0, 0)),
        ],
        out_specs=pl.BlockSpec((TB, EPAD), lambda t: (t, 0)),
        out_shape=jax.ShapeDtypeStruct((T, EPAD), jnp.float32),
    )(x, gw_pad)

    xb16 = x.astype(jnp.bfloat16)
    wg_all = jnp.concatenate([Wg, sg[None]], axis=0).astype(jnp.bfloat16)
    wu_all = jnp.concatenate([Wu, su[None]], axis=0).astype(jnp.bfloat16)
    wd_all = jnp.concatenate([Wd, sd[None]], axis=0).astype(jnp.bfloat16)

    out = pl.pallas_call(
        _moe_body,
        grid=(NT, NE, NI),
        in_specs=[
            pl.BlockSpec((TB, H), lambda t, e, i: (t, 0)),
            pl.BlockSpec((1, TI, H), lambda t, e, i: (e, i, 0)),
            pl.BlockSpec((1, TI, H), lambda t, e, i: (e, i, 0)),
            pl.BlockSpec((1, H, TI), lambda t, e, i: (e, 0, i)),
            pl.BlockSpec((TB, EPAD), lambda t, e, i: (t, 0)),
        ],
        out_specs=pl.BlockSpec((TB, H), lambda t, e, i: (t, 0)),
        out_shape=jax.ShapeDtypeStruct((T, H), jnp.float32),
    )(xb16, wg_all, wu_all, wd_all, combine)
    return out
